# baseline (speedup 1.0000x reference)
import jax
import jax.numpy as jnp
from jax import lax
from jax.experimental import pallas as pl
from jax.experimental.pallas import tpu as pltpu
from jax.experimental.pallas import tpu_sc as plsc

_E = 16
_B = 1024
_IN = 320
_H = 256
_TILE = 64
_NSLOT = 2048
_NT = _NSLOT // _TILE
_NW = 32
_SPW = _B // _NW


def _route_body(x_hbm, skill_hbm, xs_hbm, dst_hbm, eot_hbm,
                skill_v, tbl_v, snap_v, chunk_v, dst_v, rows_v, eot_v, sem):
    c = lax.axis_index("c")
    s = lax.axis_index("s")
    wid = s * 2 + c

    pltpu.sync_copy(skill_hbm, skill_v)

    ones = jnp.ones((16,), jnp.int32)
    zeros = jnp.zeros((16,), jnp.int32)

    tbl_v[...] = zeros
    my_chunk = wid * (_SPW // 16)

    def hist_step(i, carry):
        sc = skill_v[pl.ds(i * 16, 16)]
        plsc.addupdate_scatter(tbl_v, [sc], ones)
        return carry

    lax.fori_loop(0, my_chunk, hist_step, 0)
    snap_v[...] = tbl_v[...]
    lax.fori_loop(my_chunk, _B // 16, hist_step, 0)

    totals = tbl_v[...]
    prefix_mine = snap_v[...]
    cap = ((totals + (_TILE - 1)) // _TILE) * _TILE
    end_incl = plsc.cumsum(cap)
    base_excl = end_incl - cap

    tbl_v[...] = base_excl + prefix_mine
    base_i = wid * _SPW

    li = lax.iota(jnp.int32, 16)
    for k in range(_SPW // 16):
        sc = skill_v[pl.ds(base_i + k * 16, 16)]
        chunk_v[...] = sc
        base = plsc.load_gather(tbl_v, [sc])

        def win_step(j, w):
            sj = plsc.load_gather(chunk_v, [jnp.full((16,), j, jnp.int32)])
            return w + jnp.where((sc == sj) & (li > j), 1, 0)

        within = lax.fori_loop(0, 15, win_step, zeros)
        dst_v[pl.ds(k * 16, 16)] = base + within
        plsc.addupdate_scatter(tbl_v, [sc], ones)

    pltpu.sync_copy(x_hbm.at[pl.ds(base_i, _SPW)], rows_v)
    pltpu.async_copy(rows_v, xs_hbm.at[dst_v], sem).wait()
    pltpu.sync_copy(dst_v, dst_hbm.at[pl.ds(base_i, _SPW)])

    @pl.when(jnp.logical_and(c == 0, s == 0))
    def _eot():
        snap_v[...] = end_incl
        t0 = lax.iota(jnp.int32, 16) * _TILE
        t1 = (lax.iota(jnp.int32, 16) + 16) * _TILE

        def eot_step(e, accs):
            a0, a1 = accs
            endv = plsc.load_gather(snap_v, [jnp.full((16,), e, jnp.int32)])
            a0 = a0 + jnp.where(endv <= t0, 1, 0)
            a1 = a1 + jnp.where(endv <= t1, 1, 0)
            return (a0, a1)

        z = jnp.zeros((16,), jnp.int32)
        a0, a1 = lax.fori_loop(0, _E, eot_step, (z, z))
        eot_v[pl.ds(0, 16)] = jnp.minimum(a0, _E - 1)
        eot_v[pl.ds(16, 16)] = jnp.minimum(a1, _E - 1)
        pltpu.sync_copy(eot_v, eot_hbm)


_route = pl.kernel(
    _route_body,
    out_type=(
        jax.ShapeDtypeStruct((_NSLOT, _IN), jnp.float32),
        jax.ShapeDtypeStruct((_B,), jnp.int32),
        jax.ShapeDtypeStruct((_NT,), jnp.int32),
    ),
    mesh=plsc.VectorSubcoreMesh(core_axis_name="c", subcore_axis_name="s"),
    compiler_params=pltpu.CompilerParams(needs_layout_passes=False, use_tc_tiling_on_sc=False),
    scratch_types=[
        pltpu.VMEM((_B,), jnp.int32),
        pltpu.VMEM((16,), jnp.int32),
        pltpu.VMEM((16,), jnp.int32),
        pltpu.VMEM((16,), jnp.int32),
        pltpu.VMEM((_SPW,), jnp.int32),
        pltpu.VMEM((_SPW, _IN), jnp.float32),
        pltpu.VMEM((_NT,), jnp.int32),
        pltpu.SemaphoreType.DMA,
    ],
)


def _mlp_body(eot_ref, xs_ref, dst_ref, w1_ref, b1_ref, w2_ref, b2_ref,
              w3_ref, b3_ref, out_ref):
    del eot_ref
    t = pl.program_id(0)
    h = jnp.dot(xs_ref[...], w1_ref[0], preferred_element_type=jnp.float32)
    h = jnp.maximum(h + b1_ref[0], 0.0)
    h = jnp.dot(h, w2_ref[0], preferred_element_type=jnp.float32)
    h = jnp.maximum(h + b2_ref[0], 0.0)
    logit = jnp.sum(h * w3_ref[0], axis=1, keepdims=True)
    prob = jax.nn.sigmoid(logit + b3_ref[0])

    slot = t * _TILE + jax.lax.broadcasted_iota(jnp.int32, (_B, _TILE), 1)
    mask = (dst_ref[...] == slot).astype(jnp.float32)
    contrib = jnp.dot(mask, prob, preferred_element_type=jnp.float32,
                      precision=jax.lax.Precision.HIGHEST)

    @pl.when(t == 0)
    def _init():
        out_ref[...] = contrib

    @pl.when(t > 0)
    def _acc():
        out_ref[...] = out_ref[...] + contrib


def kernel(observation, action, skill_idx, W1, b1, W2, b2, W3, b3):
    x = jnp.concatenate([observation, action], axis=1)
    skill = skill_idx.astype(jnp.int32)

    xs, dst, eot = _route(x, skill)

    b1r = b1.reshape(_E, 1, _H)
    b2r = b2.reshape(_E, 1, _H)
    w3r = W3.reshape(_E, 1, _H)
    b3r = b3.reshape(_E, 1, 1)

    grid_spec = pltpu.PrefetchScalarGridSpec(
        num_scalar_prefetch=1,
        grid=(_NT,),
        in_specs=[
            pl.BlockSpec((_TILE, _IN), lambda t, eot: (t, 0)),
            pl.BlockSpec((_B, 1), lambda t, eot: (0, 0)),
            pl.BlockSpec((1, _IN, _H), lambda t, eot: (eot[t], 0, 0)),
            pl.BlockSpec((1, 1, _H), lambda t, eot: (eot[t], 0, 0)),
            pl.BlockSpec((1, _H, _H), lambda t, eot: (eot[t], 0, 0)),
            pl.BlockSpec((1, 1, _H), lambda t, eot: (eot[t], 0, 0)),
            pl.BlockSpec((1, 1, _H), lambda t, eot: (eot[t], 0, 0)),
            pl.BlockSpec((1, 1, 1), lambda t, eot: (eot[t], 0, 0)),
        ],
        out_specs=pl.BlockSpec((_B, 1), lambda t, eot: (0, 0)),
    )
    out = pl.pallas_call(
        _mlp_body,
        grid_spec=grid_spec,
        out_shape=jax.ShapeDtypeStruct((_B, 1), jnp.float32),
    )(eot, xs, dst.reshape(_B, 1), W1, b1r, W2, b2r, w3r, b3r)
    return out


# speedup vs baseline: 1.1777x; 1.1777x over previous
import jax
import jax.numpy as jnp
from jax import lax
from jax.experimental import pallas as pl
from jax.experimental.pallas import tpu as pltpu
from jax.experimental.pallas import tpu_sc as plsc

_E = 16
_B = 1024
_IN = 320
_INP = 384
_H = 256
_TILE = 64
_NSLOT = 2048
_NT = _NSLOT // _TILE
_NW = 32
_SPW = _B // _NW


def _route_body(x_hbm, skill_hbm, xs_hbm, dst_hbm, eot_hbm,
                skill_v, tbl_v, snap_v, chunk_v, dst_v, rows_v, eot_v, sem):
    c = lax.axis_index("c")
    s = lax.axis_index("s")
    wid = s * 2 + c

    pltpu.sync_copy(skill_hbm, skill_v)

    ones = jnp.ones((16,), jnp.int32)
    zeros = jnp.zeros((16,), jnp.int32)

    tbl_v[...] = zeros
    my_chunk = wid * (_SPW // 16)

    def hist_step(i, carry):
        sc = skill_v[pl.ds(i * 16, 16)]
        plsc.addupdate_scatter(tbl_v, [sc], ones)
        return carry

    lax.fori_loop(0, my_chunk, hist_step, 0)
    snap_v[...] = tbl_v[...]
    lax.fori_loop(my_chunk, _B // 16, hist_step, 0)

    totals = tbl_v[...]
    prefix_mine = snap_v[...]
    cap = ((totals + (_TILE - 1)) // _TILE) * _TILE
    end_incl = plsc.cumsum(cap)
    base_excl = end_incl - cap

    tbl_v[...] = base_excl + prefix_mine
    base_i = wid * _SPW

    li = lax.iota(jnp.int32, 16)
    for k in range(_SPW // 16):
        sc = skill_v[pl.ds(base_i + k * 16, 16)]
        chunk_v[...] = sc
        base = plsc.load_gather(tbl_v, [sc])

        def win_step(j, w):
            sj = plsc.load_gather(chunk_v, [jnp.full((16,), j, jnp.int32)])
            return w + jnp.where((sc == sj) & (li > j), 1, 0)

        within = lax.fori_loop(0, 15, win_step, zeros)
        dst_v[pl.ds(k * 16, 16)] = base + within
        plsc.addupdate_scatter(tbl_v, [sc], ones)

    pltpu.sync_copy(x_hbm.at[pl.ds(base_i, _SPW)], rows_v)
    pltpu.async_copy(rows_v, xs_hbm.at[dst_v], sem).wait()
    pltpu.sync_copy(dst_v, dst_hbm.at[pl.ds(base_i, _SPW)])

    @pl.when(jnp.logical_and(c == 0, s == 0))
    def _eot():
        snap_v[...] = end_incl
        t0 = lax.iota(jnp.int32, 16) * _TILE
        t1 = (lax.iota(jnp.int32, 16) + 16) * _TILE

        def eot_step(e, accs):
            a0, a1 = accs
            endv = plsc.load_gather(snap_v, [jnp.full((16,), e, jnp.int32)])
            a0 = a0 + jnp.where(endv <= t0, 1, 0)
            a1 = a1 + jnp.where(endv <= t1, 1, 0)
            return (a0, a1)

        z = jnp.zeros((16,), jnp.int32)
        a0, a1 = lax.fori_loop(0, _E, eot_step, (z, z))
        eot_v[pl.ds(0, 16)] = jnp.minimum(a0, _E - 1)
        eot_v[pl.ds(16, 16)] = jnp.minimum(a1, _E - 1)
        pltpu.sync_copy(eot_v, eot_hbm)


_route = pl.kernel(
    _route_body,
    out_type=(
        jax.ShapeDtypeStruct((_NSLOT, _INP), jnp.float32),
        jax.ShapeDtypeStruct((_B,), jnp.int32),
        jax.ShapeDtypeStruct((_NT,), jnp.int32),
    ),
    mesh=plsc.VectorSubcoreMesh(core_axis_name="c", subcore_axis_name="s"),
    compiler_params=pltpu.CompilerParams(needs_layout_passes=False),
    scratch_types=[
        pltpu.VMEM((_B,), jnp.int32),
        pltpu.VMEM((16,), jnp.int32),
        pltpu.VMEM((16,), jnp.int32),
        pltpu.VMEM((16,), jnp.int32),
        pltpu.VMEM((_SPW,), jnp.int32),
        pltpu.VMEM((_SPW, _INP), jnp.float32),
        pltpu.VMEM((_NT,), jnp.int32),
        pltpu.SemaphoreType.DMA,
    ],
)


def _mlp_body(eot_ref, xs_ref, dst_ref, w1_ref, b1_ref, w2_ref, b2_ref,
              w3_ref, b3_ref, out_ref, ys_ref):
    t = pl.program_id(0)
    del eot_ref

    @pl.when(t < _NT)
    def _mlp():
        x = xs_ref[...][:, :_IN]
        h = jnp.dot(x, w1_ref[0], preferred_element_type=jnp.float32)
        h = jnp.maximum(h + b1_ref[0], 0.0)
        h = jnp.dot(h, w2_ref[0], preferred_element_type=jnp.float32)
        h = jnp.maximum(h + b2_ref[0], 0.0)
        logit = jnp.sum(h * w3_ref[0], axis=1, keepdims=True)
        ys_ref[pl.ds(t * _TILE, _TILE), :] = jax.nn.sigmoid(logit + b3_ref[0])

    @pl.when(t == _NT)
    def _unpermute():
        dst = dst_ref[...]
        acc = jnp.zeros((_B, 1), jnp.float32)
        for k in range(_NSLOT // 128):
            slot = k * 128 + lax.broadcasted_iota(jnp.int32, (_B, 128), 1)
            mask = (dst == slot).astype(jnp.float32)
            acc = acc + jnp.dot(mask, ys_ref[pl.ds(k * 128, 128), :],
                                preferred_element_type=jnp.float32,
                                precision=lax.Precision.HIGHEST)
        out_ref[...] = acc


def kernel(observation, action, skill_idx, W1, b1, W2, b2, W3, b3):
    x = jnp.concatenate(
        [observation, action,
         jnp.zeros((_B, _INP - _IN), jnp.float32)], axis=1)
    skill = skill_idx.astype(jnp.int32)

    xs, dst, eot = _route(x, skill)

    b1r = b1.reshape(_E, 1, _H)
    b2r = b2.reshape(_E, 1, _H)
    w3r = W3.reshape(_E, 1, _H)
    b3r = b3.reshape(_E, 1, 1)

    grid_spec = pltpu.PrefetchScalarGridSpec(
        num_scalar_prefetch=1,
        grid=(_NT + 1,),
        in_specs=[
            pl.BlockSpec((_TILE, _INP),
                         lambda t, eot: (jnp.minimum(t, _NT - 1), 0)),
            pl.BlockSpec((_B, 1), lambda t, eot: (0, 0)),
            pl.BlockSpec((1, _IN, _H),
                         lambda t, eot: (eot[jnp.minimum(t, _NT - 1)], 0, 0)),
            pl.BlockSpec((1, 1, _H),
                         lambda t, eot: (eot[jnp.minimum(t, _NT - 1)], 0, 0)),
            pl.BlockSpec((1, _H, _H),
                         lambda t, eot: (eot[jnp.minimum(t, _NT - 1)], 0, 0)),
            pl.BlockSpec((1, 1, _H),
                         lambda t, eot: (eot[jnp.minimum(t, _NT - 1)], 0, 0)),
            pl.BlockSpec((1, 1, _H),
                         lambda t, eot: (eot[jnp.minimum(t, _NT - 1)], 0, 0)),
            pl.BlockSpec((1, 1, 1),
                         lambda t, eot: (eot[jnp.minimum(t, _NT - 1)], 0, 0)),
        ],
        out_specs=pl.BlockSpec((_B, 1), lambda t, eot: (0, 0)),
        scratch_shapes=[pltpu.VMEM((_NSLOT, 1), jnp.float32)],
    )
    out = pl.pallas_call(
        _mlp_body,
        grid_spec=grid_spec,
        out_shape=jax.ShapeDtypeStruct((_B, 1), jnp.float32),
    )(eot, xs, dst.reshape(_B, 1), W1, b1r, W2, b2r, w3r, b3r)
    return out


# speedup vs baseline: 2.3572x; 2.0016x over previous
import jax
import jax.numpy as jnp
from jax.experimental import pallas as pl

_NUM_SKILLS = 16


def _mlp_body(x_ref, skill_ref, w1_ref, b1_ref, w2_ref, b2_ref, w3_ref,
              b3_ref, out_ref):
    e = pl.program_id(0)
    w1 = w1_ref[0].astype(jnp.bfloat16)
    w2 = w2_ref[0].astype(jnp.bfloat16)
    h = jnp.dot(x_ref[...], w1, preferred_element_type=jnp.float32)
    h = jnp.maximum(h + b1_ref[0], 0.0)
    h = jnp.dot(h.astype(jnp.bfloat16), w2, preferred_element_type=jnp.float32)
    h = jnp.maximum(h + b2_ref[0], 0.0)
    logit = jnp.sum(h * w3_ref[0], axis=1, keepdims=True)
    prob = jax.nn.sigmoid(logit + b3_ref[0])
    contrib = jnp.where(skill_ref[...] == e, prob, 0.0)

    @pl.when(e == 0)
    def _init():
        out_ref[...] = contrib

    @pl.when(e > 0)
    def _acc():
        out_ref[...] = out_ref[...] + contrib


def kernel(observation, action, skill_idx, W1, b1, W2, b2, W3, b3):
    batch = observation.shape[0]
    in_dim = observation.shape[1] + action.shape[1]
    h1 = W1.shape[2]
    h2 = W2.shape[2]

    x = jnp.concatenate([observation, action], axis=1).astype(jnp.bfloat16)
    skill = skill_idx.astype(jnp.int32).reshape(batch, 1)
    b1r = b1.reshape(_NUM_SKILLS, 1, h1)
    b2r = b2.reshape(_NUM_SKILLS, 1, h2)
    w3 = W3.reshape(_NUM_SKILLS, 1, h2)
    b3r = b3.reshape(_NUM_SKILLS, 1, 1)

    out = pl.pallas_call(
        _mlp_body,
        grid=(_NUM_SKILLS,),
        in_specs=[
            pl.BlockSpec((batch, in_dim), lambda e: (0, 0)),
            pl.BlockSpec((batch, 1), lambda e: (0, 0)),
            pl.BlockSpec((1, in_dim, h1), lambda e: (e, 0, 0)),
            pl.BlockSpec((1, 1, h1), lambda e: (e, 0, 0)),
            pl.BlockSpec((1, h1, h2), lambda e: (e, 0, 0)),
            pl.BlockSpec((1, 1, h2), lambda e: (e, 0, 0)),
            pl.BlockSpec((1, 1, h2), lambda e: (e, 0, 0)),
            pl.BlockSpec((1, 1, 1), lambda e: (e, 0, 0)),
        ],
        out_specs=pl.BlockSpec((batch, 1), lambda e: (0, 0)),
        out_shape=jax.ShapeDtypeStruct((batch, 1), jnp.float32),
    )(x, skill, W1, b1r, W2, b2r, w3, b3r)
    return out


# speedup vs baseline: 2.6601x; 1.1285x over previous
import jax
import jax.numpy as jnp
from jax.experimental import pallas as pl

_NUM_SKILLS = 16


def _mlp_body(x_ref, skill_ref, w1_ref, b1_ref, w2_ref, b2_ref, w3_ref,
              b3_ref, out_ref):
    x = x_ref[...]
    skill = skill_ref[...]
    acc = jnp.zeros(out_ref.shape, jnp.float32)
    for e in range(_NUM_SKILLS):
        h = jnp.dot(x, w1_ref[e], preferred_element_type=jnp.float32)
        h = jnp.maximum(h + b1_ref[e], 0.0)
        h = jnp.dot(h, w2_ref[e], preferred_element_type=jnp.float32)
        h = jnp.maximum(h + b2_ref[e], 0.0)
        logit = jnp.sum(h * w3_ref[e], axis=1, keepdims=True)
        prob = jax.nn.sigmoid(logit + b3_ref[e])
        acc = acc + jnp.where(skill == e, prob, 0.0)
    out_ref[...] = acc


def kernel(observation, action, skill_idx, W1, b1, W2, b2, W3, b3):
    batch = observation.shape[0]
    h1 = W1.shape[2]
    h2 = W2.shape[2]

    x = jnp.concatenate([observation, action], axis=1)
    skill = skill_idx.astype(jnp.int32).reshape(batch, 1)
    b1r = b1.reshape(_NUM_SKILLS, 1, h1)
    b2r = b2.reshape(_NUM_SKILLS, 1, h2)
    w3 = W3.reshape(_NUM_SKILLS, 1, h2)
    b3r = b3.reshape(_NUM_SKILLS, 1, 1)

    out = pl.pallas_call(
        _mlp_body,
        out_shape=jax.ShapeDtypeStruct((batch, 1), jnp.float32),
    )(x, skill, W1, b1r, W2, b2r, w3, b3r)
    return out
